# baseline (speedup 1.0000x reference)
import functools

import jax
import jax.numpy as jnp
from jax import lax
from jax.experimental import pallas as pl
from jax.experimental.pallas import tpu as pltpu
from jax.experimental.pallas import tpu_sc as plsc

NN = 10000
EE = 320000
FF = 128
CC = 16

NTILES = 32
EPT = EE // NTILES
FPT = FF // NTILES
CH = 8000
NCH = EE // CH

_mesh = plsc.VectorSubcoreMesh(core_axis_name="c", subcore_axis_name="s")
_sc_params = pltpu.CompilerParams(use_tc_tiling_on_sc=False,
                                  needs_layout_passes=False)


def _wid():
    return lax.axis_index("s") * 2 + lax.axis_index("c")


def _zero_1d(ref, n):
    z = jnp.zeros((16,), jnp.float32)

    @plsc.parallel_loop(0, n // 16, unroll=8)
    def _(i):
        ref[pl.ds(i * 16, 16)] = z


@functools.partial(
    pl.kernel,
    out_type=(
        jax.ShapeDtypeStruct((EE,), jnp.int32),
        jax.ShapeDtypeStruct((NTILES, NN), jnp.float32),
        jax.ShapeDtypeStruct((NTILES, NN), jnp.float32),
    ),
    mesh=_mesh,
    scratch_types=[
        pltpu.VMEM((EPT,), jnp.int32),
        pltpu.VMEM((EPT,), jnp.int32),
        pltpu.VMEM((EPT,), jnp.int32),
        pltpu.VMEM((NN,), jnp.float32),
        pltpu.VMEM((NN,), jnp.float32),
    ],
    compiler_params=_sc_params,
)
def _sc_pack_deg(edge_hbm, ep_hbm, dego_hbm, degi_hbm,
                 srcv, dstv, epv, ho, hi):
    w = _wid()
    base = w * EPT
    pltpu.sync_copy(edge_hbm.at[0, pl.ds(base, EPT)], srcv)
    pltpu.sync_copy(edge_hbm.at[1, pl.ds(base, EPT)], dstv)
    _zero_1d(ho, NN)
    _zero_1d(hi, NN)
    ones = jnp.full((16,), 1.0, jnp.float32)

    @plsc.parallel_loop(0, EPT // 16, unroll=5)
    def _(j):
        s = srcv[pl.ds(j * 16, 16)]
        d = dstv[pl.ds(j * 16, 16)]
        epv[pl.ds(j * 16, 16)] = s | (d << 16)
        plsc.addupdate_scatter(ho, [s], ones)
        plsc.addupdate_scatter(hi, [d], ones)
    pltpu.sync_copy(epv, ep_hbm.at[pl.ds(base, EPT)])
    pltpu.sync_copy(ho, dego_hbm.at[w])
    pltpu.sync_copy(hi, degi_hbm.at[w])


def _make_sc_layer(with_wsum):
    out_type = [jax.ShapeDtypeStruct((FF, NN), jnp.float32)]
    scratch = (
        [pltpu.VMEM((NN,), jnp.int32)] * 2
        + [pltpu.VMEM((NN,), jnp.float32)] * FPT
        + [
            pltpu.VMEM((CH,), jnp.int32),
            pltpu.VMEM((CH,), jnp.int32),
            pltpu.SemaphoreType.DMA,
            pltpu.SemaphoreType.DMA,
        ]
    )
    if with_wsum:
        out_type.append(jax.ShapeDtypeStruct((NTILES, NN), jnp.float32))
        scratch += [
            pltpu.VMEM((NN,), jnp.float32),
            pltpu.VMEM((NN,), jnp.float32),
        ]

    def body(hp_hbm, ep_hbm, *rest):
        if with_wsum:
            (normd_hbm, aggT_hbm, wsum_hbm, hp0, hp1, a0, a1, a2, a3,
             ebA, ebB, semA, semB, nd_l, ws_l) = rest
        else:
            (aggT_hbm, hp0, hp1, a0, a1, a2, a3,
             ebA, ebB, semA, semB) = rest
        hp_refs = (hp0, hp1)
        a_refs = (a0, a1, a2, a3)
        w = _wid()
        for k in range(2):
            pltpu.sync_copy(hp_hbm.at[2 * w + k], hp_refs[k])
        if with_wsum:
            pltpu.sync_copy(normd_hbm, nd_l)
            _zero_1d(ws_l, NN)
        for f in range(FPT):
            _zero_1d(a_refs[f], NN)
        himask = jnp.int32(-65536)

        def start(cidx, buf, sem):
            return pltpu.async_copy(ep_hbm.at[pl.ds(cidx * CH, CH)], buf, sem)

        def wait(buf, sem):
            pltpu.make_async_copy(ep_hbm.at[pl.ds(0, CH)], buf, sem).wait()

        def do_edges(ebuf, j, wsum):
            ep = ebuf[pl.ds(j * 16, 16)]
            s = ep & 0xFFFF
            d = ep >> 16
            for k in range(2):
                g = plsc.load_gather(hp_refs[k], [s])
                vlo = plsc.bitcast(g << 16, jnp.float32)
                vhi = plsc.bitcast(g & himask, jnp.float32)
                plsc.addupdate_scatter(a_refs[2 * k], [d], vlo)
                plsc.addupdate_scatter(a_refs[2 * k + 1], [d], vhi)
            if wsum:
                nd = plsc.load_gather(nd_l, [d])
                plsc.addupdate_scatter(ws_l, [s], nd)

        def edges_plain(ebuf):
            @plsc.parallel_loop(0, CH // 16, unroll=3)
            def _(j):
                do_edges(ebuf, j, False)

        def edges_wsum(ebuf):
            @plsc.parallel_loop(0, CH // 16, unroll=3)
            def _(j):
                do_edges(ebuf, j, True)

        def process(cidx, ebuf):
            if with_wsum:
                mine = (cidx % NTILES) == w

                @pl.when(mine)
                def _():
                    edges_wsum(ebuf)

                @pl.when(jnp.logical_not(mine))
                def _():
                    edges_plain(ebuf)
            else:
                edges_plain(ebuf)

        start(0, ebA, semA)
        start(1, ebB, semB)

        def outer(i, _):
            wait(ebA, semA)

            @pl.when(i < NCH // 2 - 1)
            def _():
                start(2 * i + 2, ebA, semA)

            process(2 * i, ebA)
            wait(ebB, semB)

            @pl.when(i < NCH // 2 - 1)
            def _():
                start(2 * i + 3, ebB, semB)

            process(2 * i + 1, ebB)
            return 0

        lax.fori_loop(0, NCH // 2, outer, 0)

        pltpu.sync_copy(a_refs[0], aggT_hbm.at[2 * w])
        pltpu.sync_copy(a_refs[1], aggT_hbm.at[64 + 2 * w])
        pltpu.sync_copy(a_refs[2], aggT_hbm.at[2 * w + 1])
        pltpu.sync_copy(a_refs[3], aggT_hbm.at[64 + 2 * w + 1])
        if with_wsum:
            pltpu.sync_copy(ws_l, wsum_hbm.at[w])

    return pl.kernel(
        body,
        out_type=tuple(out_type) if with_wsum else out_type[0],
        mesh=_mesh,
        scratch_types=scratch,
        compiler_params=_sc_params,
    )


_sc_layer_wsum = _make_sc_layer(True)
_sc_layer = _make_sc_layer(False)


def _pack_bf16_pairs(h):
    lo = h[:64, :].astype(jnp.bfloat16)
    hi = h[64:, :].astype(jnp.bfloat16)
    lob = lax.bitcast_convert_type(lo, jnp.uint16).astype(jnp.uint32)
    hib = lax.bitcast_convert_type(hi, jnp.uint16).astype(jnp.uint32)
    return lax.bitcast_convert_type(lob | (hib << 16), jnp.int32)


def _tcA_body(inT, dego, degi, h_out, ns_out, nd_out):
    deg_o = jnp.sum(dego[...], axis=0, keepdims=True)
    deg_i = jnp.sum(degi[...], axis=0, keepdims=True)
    ns = jnp.where(deg_o > 0, lax.rsqrt(jnp.maximum(deg_o, 1.0)), 0.0)
    nd = jnp.where(deg_i > 0, lax.rsqrt(jnp.maximum(deg_i, 1.0)), 0.0)
    ns_out[...] = ns
    nd_out[...] = nd
    h_out[...] = _pack_bf16_pairs(inT[...] * ns)


def _tc_norms(inT, dego_p, degi_p):
    return pl.pallas_call(
        _tcA_body,
        out_shape=[
            jax.ShapeDtypeStruct((FF // 2, NN), jnp.int32),
            jax.ShapeDtypeStruct((1, NN), jnp.float32),
            jax.ShapeDtypeStruct((1, NN), jnp.float32),
        ],
    )(inT, dego_p, degi_p)


def _tcB_body(agg, nd, ns, W, b, h_out):
    x = agg[...] * nd[...]
    y = lax.dot_general(W[...], x, (((0,), (0,)), ((), ())),
                        preferred_element_type=jnp.float32)
    h_out[...] = _pack_bf16_pairs(jnp.maximum(y + b[...], 0.0) * ns[...])


def _tc_layer(aggT, nd, ns, W, b2d):
    return pl.pallas_call(
        _tcB_body,
        out_shape=jax.ShapeDtypeStruct((FF // 2, NN), jnp.int32),
    )(aggT, nd, ns, W, b2d)


def _tcC_body(agg, nd, wsp, ns, W2, b2, W3, b3, out):
    x = agg[...] * nd[...]
    y = lax.dot_general(W2[...], x, (((0,), (0,)), ((), ())),
                        preferred_element_type=jnp.float32)
    h2 = jnp.maximum(y + b2[...], 0.0)
    wv = ns[...] * jnp.sum(wsp[...], axis=0, keepdims=True)
    g = lax.dot_general(h2, wv, (((1,), (1,)), ((), ())),
                        preferred_element_type=jnp.float32) * (1.0 / NN)
    p = lax.dot_general(W3[...], g, (((0,), (0,)), ((), ())),
                        preferred_element_type=jnp.float32) + b3[...]
    m = jnp.max(p, axis=0, keepdims=True)
    e = jnp.exp(p - m)
    out[...] = e / jnp.sum(e, axis=0, keepdims=True)


def _tc_final(aggT, nd, wsum_p, ns, W2, b2d, W3, b3d):
    return pl.pallas_call(
        _tcC_body,
        out_shape=jax.ShapeDtypeStruct((CC, 1), jnp.float32),
    )(aggT, nd, wsum_p, ns, W2, b2d, W3, b3d)


def kernel(in_feat, edge_index, W1, b1, W2, b2, W3, b3):
    ep, dego_p, degi_p = _sc_pack_deg(edge_index)
    inT = in_feat.T
    h0sT, ns, nd = _tc_norms(inT, dego_p, degi_p)
    aggT1, wsum_p = _sc_layer_wsum(h0sT, ep, nd.reshape(NN))
    h1sT = _tc_layer(aggT1, nd, ns, W1, b1.reshape(FF, 1))
    aggT2 = _sc_layer(h1sT, ep)
    out = _tc_final(aggT2, nd, wsum_p, ns, W2, b2.reshape(FF, 1),
                    W3, b3.reshape(CC, 1))
    return out.reshape(CC)


# speedup vs baseline: 1.0042x; 1.0042x over previous
import functools

import jax
import jax.numpy as jnp
from jax import lax
from jax.experimental import pallas as pl
from jax.experimental.pallas import tpu as pltpu
from jax.experimental.pallas import tpu_sc as plsc

NN = 10000
EE = 320000
FF = 128
CC = 16

NTILES = 32
EPT = EE // NTILES
FPT = FF // NTILES
CH = 4000
NCH = EE // CH

_mesh = plsc.VectorSubcoreMesh(core_axis_name="c", subcore_axis_name="s")
_sc_params = pltpu.CompilerParams(use_tc_tiling_on_sc=False,
                                  needs_layout_passes=False,
                                  disable_bounds_checks=True)


def _wid():
    return lax.axis_index("s") * 2 + lax.axis_index("c")


def _zero_1d(ref, n):
    z = jnp.zeros((16,), jnp.float32)

    @plsc.parallel_loop(0, n // 16, unroll=8)
    def _(i):
        ref[pl.ds(i * 16, 16)] = z


@functools.partial(
    pl.kernel,
    out_type=(
        jax.ShapeDtypeStruct((EE,), jnp.int32),
        jax.ShapeDtypeStruct((NTILES, NN), jnp.float32),
        jax.ShapeDtypeStruct((NTILES, NN), jnp.float32),
    ),
    mesh=_mesh,
    scratch_types=[
        pltpu.VMEM((EPT,), jnp.int32),
        pltpu.VMEM((EPT,), jnp.int32),
        pltpu.VMEM((EPT,), jnp.int32),
        pltpu.VMEM((NN,), jnp.float32),
        pltpu.VMEM((NN,), jnp.float32),
    ],
    compiler_params=_sc_params,
)
def _sc_pack_deg(edge_hbm, ep_hbm, dego_hbm, degi_hbm,
                 srcv, dstv, epv, ho, hi):
    w = _wid()
    base = w * EPT
    pltpu.sync_copy(edge_hbm.at[0, pl.ds(base, EPT)], srcv)
    pltpu.sync_copy(edge_hbm.at[1, pl.ds(base, EPT)], dstv)
    _zero_1d(ho, NN)
    _zero_1d(hi, NN)
    ones = jnp.full((16,), 1.0, jnp.float32)

    @plsc.parallel_loop(0, EPT // 16, unroll=5)
    def _(j):
        s = srcv[pl.ds(j * 16, 16)]
        d = dstv[pl.ds(j * 16, 16)]
        epv[pl.ds(j * 16, 16)] = s | (d << 16)
        plsc.addupdate_scatter(ho, [s], ones)
        plsc.addupdate_scatter(hi, [d], ones)
    pltpu.sync_copy(epv, ep_hbm.at[pl.ds(base, EPT)])
    pltpu.sync_copy(ho, dego_hbm.at[w])
    pltpu.sync_copy(hi, degi_hbm.at[w])


def _make_sc_layer(with_wsum):
    out_type = [jax.ShapeDtypeStruct((FF, NN), jnp.float32)]
    scratch = (
        [pltpu.VMEM((NN,), jnp.int32)] * 2
        + [pltpu.VMEM((NN,), jnp.float32)] * FPT
        + [
            pltpu.VMEM((CH,), jnp.int32),
            pltpu.VMEM((CH,), jnp.int32),
            pltpu.SemaphoreType.DMA,
            pltpu.SemaphoreType.DMA,
        ]
    )
    if with_wsum:
        out_type.append(jax.ShapeDtypeStruct((NTILES, NN), jnp.float32))
        scratch += [
            pltpu.VMEM((NN,), jnp.float32),
            pltpu.VMEM((NN,), jnp.float32),
        ]

    def body(hp_hbm, ep_hbm, *rest):
        if with_wsum:
            (normd_hbm, aggT_hbm, wsum_hbm, hp0, hp1, a0, a1, a2, a3,
             ebA, ebB, semA, semB, nd_l, ws_l) = rest
        else:
            (aggT_hbm, hp0, hp1, a0, a1, a2, a3,
             ebA, ebB, semA, semB) = rest
        hp_refs = (hp0, hp1)
        a_refs = (a0, a1, a2, a3)
        w = _wid()
        for k in range(2):
            pltpu.sync_copy(hp_hbm.at[2 * w + k], hp_refs[k])
        if with_wsum:
            pltpu.sync_copy(normd_hbm, nd_l)
            _zero_1d(ws_l, NN)
        for f in range(FPT):
            _zero_1d(a_refs[f], NN)
        himask = jnp.int32(-65536)

        def start(cidx, buf, sem):
            return pltpu.async_copy(ep_hbm.at[pl.ds(cidx * CH, CH)], buf, sem)

        def wait(buf, sem):
            pltpu.make_async_copy(ep_hbm.at[pl.ds(0, CH)], buf, sem).wait()

        def do_edges(ebuf, j, wsum):
            ep = ebuf[pl.ds(j * 16, 16)]
            s = ep & 0xFFFF
            d = ep >> 16
            for k in range(2):
                g = plsc.load_gather(hp_refs[k], [s])
                vlo = plsc.bitcast(g << 16, jnp.float32)
                vhi = plsc.bitcast(g & himask, jnp.float32)
                plsc.addupdate_scatter(a_refs[2 * k], [d], vlo)
                plsc.addupdate_scatter(a_refs[2 * k + 1], [d], vhi)
            if wsum:
                nd = plsc.load_gather(nd_l, [d])
                plsc.addupdate_scatter(ws_l, [s], nd)

        def edges_plain(ebuf):
            @plsc.parallel_loop(0, CH // 16, unroll=3)
            def _(j):
                do_edges(ebuf, j, False)

        def edges_wsum(ebuf):
            @plsc.parallel_loop(0, CH // 16, unroll=3)
            def _(j):
                do_edges(ebuf, j, True)

        def process(cidx, ebuf):
            if with_wsum:
                mine = (cidx % NTILES) == w

                @pl.when(mine)
                def _():
                    edges_wsum(ebuf)

                @pl.when(jnp.logical_not(mine))
                def _():
                    edges_plain(ebuf)
            else:
                edges_plain(ebuf)

        start(0, ebA, semA)
        start(1, ebB, semB)

        def outer(i, _):
            wait(ebA, semA)

            @pl.when(i < NCH // 2 - 1)
            def _():
                start(2 * i + 2, ebA, semA)

            process(2 * i, ebA)
            wait(ebB, semB)

            @pl.when(i < NCH // 2 - 1)
            def _():
                start(2 * i + 3, ebB, semB)

            process(2 * i + 1, ebB)
            return 0

        lax.fori_loop(0, NCH // 2, outer, 0)

        pltpu.sync_copy(a_refs[0], aggT_hbm.at[2 * w])
        pltpu.sync_copy(a_refs[1], aggT_hbm.at[64 + 2 * w])
        pltpu.sync_copy(a_refs[2], aggT_hbm.at[2 * w + 1])
        pltpu.sync_copy(a_refs[3], aggT_hbm.at[64 + 2 * w + 1])
        if with_wsum:
            pltpu.sync_copy(ws_l, wsum_hbm.at[w])

    return pl.kernel(
        body,
        out_type=tuple(out_type) if with_wsum else out_type[0],
        mesh=_mesh,
        scratch_types=scratch,
        compiler_params=_sc_params,
    )


_sc_layer_wsum = _make_sc_layer(True)
_sc_layer = _make_sc_layer(False)


def _pack_bf16_pairs(h):
    lo = h[:64, :].astype(jnp.bfloat16)
    hi = h[64:, :].astype(jnp.bfloat16)
    lob = lax.bitcast_convert_type(lo, jnp.uint16).astype(jnp.uint32)
    hib = lax.bitcast_convert_type(hi, jnp.uint16).astype(jnp.uint32)
    return lax.bitcast_convert_type(lob | (hib << 16), jnp.int32)


def _tcA_body(inT, dego, degi, h_out, ns_out, nd_out):
    deg_o = jnp.sum(dego[...], axis=0, keepdims=True)
    deg_i = jnp.sum(degi[...], axis=0, keepdims=True)
    ns = jnp.where(deg_o > 0, lax.rsqrt(jnp.maximum(deg_o, 1.0)), 0.0)
    nd = jnp.where(deg_i > 0, lax.rsqrt(jnp.maximum(deg_i, 1.0)), 0.0)
    ns_out[...] = ns
    nd_out[...] = nd
    h_out[...] = _pack_bf16_pairs(inT[...] * ns)


def _tc_norms(inT, dego_p, degi_p):
    return pl.pallas_call(
        _tcA_body,
        out_shape=[
            jax.ShapeDtypeStruct((FF // 2, NN), jnp.int32),
            jax.ShapeDtypeStruct((1, NN), jnp.float32),
            jax.ShapeDtypeStruct((1, NN), jnp.float32),
        ],
    )(inT, dego_p, degi_p)


def _tcB_body(agg, nd, ns, W, b, h_out):
    x = agg[...] * nd[...]
    y = lax.dot_general(W[...], x, (((0,), (0,)), ((), ())),
                        preferred_element_type=jnp.float32)
    h_out[...] = _pack_bf16_pairs(jnp.maximum(y + b[...], 0.0) * ns[...])


def _tc_layer(aggT, nd, ns, W, b2d):
    return pl.pallas_call(
        _tcB_body,
        out_shape=jax.ShapeDtypeStruct((FF // 2, NN), jnp.int32),
    )(aggT, nd, ns, W, b2d)


def _tcC_body(agg, nd, wsp, ns, W2, b2, W3, b3, out):
    x = agg[...] * nd[...]
    y = lax.dot_general(W2[...], x, (((0,), (0,)), ((), ())),
                        preferred_element_type=jnp.float32)
    h2 = jnp.maximum(y + b2[...], 0.0)
    wv = ns[...] * jnp.sum(wsp[...], axis=0, keepdims=True)
    g = lax.dot_general(h2, wv, (((1,), (1,)), ((), ())),
                        preferred_element_type=jnp.float32) * (1.0 / NN)
    p = lax.dot_general(W3[...], g, (((0,), (0,)), ((), ())),
                        preferred_element_type=jnp.float32) + b3[...]
    m = jnp.max(p, axis=0, keepdims=True)
    e = jnp.exp(p - m)
    out[...] = e / jnp.sum(e, axis=0, keepdims=True)


def _tc_final(aggT, nd, wsum_p, ns, W2, b2d, W3, b3d):
    return pl.pallas_call(
        _tcC_body,
        out_shape=jax.ShapeDtypeStruct((CC, 1), jnp.float32),
    )(aggT, nd, wsum_p, ns, W2, b2d, W3, b3d)


def kernel(in_feat, edge_index, W1, b1, W2, b2, W3, b3):
    ep, dego_p, degi_p = _sc_pack_deg(edge_index)
    inT = in_feat.T
    h0sT, ns, nd = _tc_norms(inT, dego_p, degi_p)
    aggT1, wsum_p = _sc_layer_wsum(h0sT, ep, nd.reshape(NN))
    h1sT = _tc_layer(aggT1, nd, ns, W1, b1.reshape(FF, 1))
    aggT2 = _sc_layer(h1sT, ep)
    out = _tc_final(aggT2, nd, wsum_p, ns, W2, b2.reshape(FF, 1),
                    W3, b3.reshape(CC, 1))
    return out.reshape(CC)


# speedup vs baseline: 1.0051x; 1.0008x over previous
import functools

import jax
import jax.numpy as jnp
from jax import lax
from jax.experimental import pallas as pl
from jax.experimental.pallas import tpu as pltpu
from jax.experimental.pallas import tpu_sc as plsc

NN = 10000
EE = 320000
FF = 128
CC = 16

NTILES = 32
EPT = EE // NTILES
FPT = FF // NTILES
CH = 4000
NCH = EE // CH

_mesh = plsc.VectorSubcoreMesh(core_axis_name="c", subcore_axis_name="s")
_sc_params = pltpu.CompilerParams(use_tc_tiling_on_sc=False,
                                  needs_layout_passes=False,
                                  disable_bounds_checks=True)


def _wid():
    return lax.axis_index("s") * 2 + lax.axis_index("c")


def _zero_1d(ref, n):
    z = jnp.zeros((16,), jnp.float32)

    @plsc.parallel_loop(0, n // 16, unroll=8)
    def _(i):
        ref[pl.ds(i * 16, 16)] = z


@functools.partial(
    pl.kernel,
    out_type=(
        jax.ShapeDtypeStruct((EE,), jnp.int32),
        jax.ShapeDtypeStruct((NTILES, NN), jnp.float32),
        jax.ShapeDtypeStruct((NTILES, NN), jnp.float32),
    ),
    mesh=_mesh,
    scratch_types=[
        pltpu.VMEM((EPT,), jnp.int32),
        pltpu.VMEM((EPT,), jnp.int32),
        pltpu.VMEM((EPT,), jnp.int32),
        pltpu.VMEM((NN,), jnp.float32),
        pltpu.VMEM((NN,), jnp.float32),
    ],
    compiler_params=_sc_params,
)
def _sc_pack_deg(edge_hbm, ep_hbm, dego_hbm, degi_hbm,
                 srcv, dstv, epv, ho, hi):
    w = _wid()
    base = w * EPT
    pltpu.sync_copy(edge_hbm.at[0, pl.ds(base, EPT)], srcv)
    pltpu.sync_copy(edge_hbm.at[1, pl.ds(base, EPT)], dstv)
    _zero_1d(ho, NN)
    _zero_1d(hi, NN)
    ones = jnp.full((16,), 1.0, jnp.float32)

    @plsc.parallel_loop(0, EPT // 16, unroll=5)
    def _(j):
        s = srcv[pl.ds(j * 16, 16)]
        d = dstv[pl.ds(j * 16, 16)]
        epv[pl.ds(j * 16, 16)] = s | (d << 16)
        plsc.addupdate_scatter(ho, [s], ones)
        plsc.addupdate_scatter(hi, [d], ones)
    pltpu.sync_copy(epv, ep_hbm.at[pl.ds(base, EPT)])
    pltpu.sync_copy(ho, dego_hbm.at[w])
    pltpu.sync_copy(hi, degi_hbm.at[w])


def _make_sc_layer(with_wsum):
    out_type = [jax.ShapeDtypeStruct((FF, NN), jnp.float32)]
    scratch = (
        [pltpu.VMEM((NN,), jnp.int32)] * 2
        + [pltpu.VMEM((NN,), jnp.float32)] * FPT
        + [
            pltpu.VMEM((CH,), jnp.int32),
            pltpu.VMEM((CH,), jnp.int32),
            pltpu.SemaphoreType.DMA,
            pltpu.SemaphoreType.DMA,
        ]
    )
    if with_wsum:
        out_type.append(jax.ShapeDtypeStruct((NTILES, NN), jnp.float32))
        scratch += [
            pltpu.VMEM((NN,), jnp.float32),
            pltpu.VMEM((NN,), jnp.float32),
        ]

    def body(hp_hbm, ep_hbm, *rest):
        if with_wsum:
            (normd_hbm, aggT_hbm, wsum_hbm, hp0, hp1, a0, a1, a2, a3,
             ebA, ebB, semA, semB, nd_l, ws_l) = rest
        else:
            (aggT_hbm, hp0, hp1, a0, a1, a2, a3,
             ebA, ebB, semA, semB) = rest
        hp_refs = (hp0, hp1)
        a_refs = (a0, a1, a2, a3)
        w = _wid()
        for k in range(2):
            pltpu.sync_copy(hp_hbm.at[2 * w + k], hp_refs[k])
        if with_wsum:
            pltpu.sync_copy(normd_hbm, nd_l)
            _zero_1d(ws_l, NN)
        for f in range(FPT):
            _zero_1d(a_refs[f], NN)
        himask = jnp.int32(-65536)

        def start(cidx, buf, sem):
            return pltpu.async_copy(ep_hbm.at[pl.ds(cidx * CH, CH)], buf, sem)

        def wait(buf, sem):
            pltpu.make_async_copy(ep_hbm.at[pl.ds(0, CH)], buf, sem).wait()

        def do_edges(ebuf, j, wsum):
            ep = ebuf[pl.ds(j * 16, 16)]
            s = ep & 0xFFFF
            d = ep >> 16
            for k in range(2):
                g = plsc.load_gather(hp_refs[k], [s])
                vlo = plsc.bitcast(g << 16, jnp.float32)
                vhi = plsc.bitcast(g & himask, jnp.float32)
                plsc.addupdate_scatter(a_refs[2 * k], [d], vlo)
                plsc.addupdate_scatter(a_refs[2 * k + 1], [d], vhi)
            if wsum:
                nd = plsc.load_gather(nd_l, [d])
                plsc.addupdate_scatter(ws_l, [s], nd)

        def edges_plain(ebuf):
            @plsc.parallel_loop(0, CH // 16, unroll=3)
            def _(j):
                do_edges(ebuf, j, False)

        def edges_wsum(ebuf):
            @plsc.parallel_loop(0, CH // 16, unroll=3)
            def _(j):
                do_edges(ebuf, j, True)

        def process(cidx, ebuf):
            if with_wsum:
                mine = (cidx % NTILES) == w

                @pl.when(mine)
                def _():
                    edges_wsum(ebuf)

                @pl.when(jnp.logical_not(mine))
                def _():
                    edges_plain(ebuf)
            else:
                edges_plain(ebuf)

        start(0, ebA, semA)
        start(1, ebB, semB)

        def outer(i, _):
            wait(ebA, semA)

            @pl.when(i < NCH // 2 - 1)
            def _():
                start(2 * i + 2, ebA, semA)

            process(2 * i, ebA)
            wait(ebB, semB)

            @pl.when(i < NCH // 2 - 1)
            def _():
                start(2 * i + 3, ebB, semB)

            process(2 * i + 1, ebB)
            return 0

        lax.fori_loop(0, NCH // 2, outer, 0)

        pltpu.sync_copy(a_refs[0], aggT_hbm.at[2 * w])
        pltpu.sync_copy(a_refs[1], aggT_hbm.at[64 + 2 * w])
        pltpu.sync_copy(a_refs[2], aggT_hbm.at[2 * w + 1])
        pltpu.sync_copy(a_refs[3], aggT_hbm.at[64 + 2 * w + 1])
        if with_wsum:
            pltpu.sync_copy(ws_l, wsum_hbm.at[w])

    return pl.kernel(
        body,
        out_type=tuple(out_type) if with_wsum else out_type[0],
        mesh=_mesh,
        scratch_types=scratch,
        compiler_params=_sc_params,
    )


_sc_layer_wsum = _make_sc_layer(True)
_sc_layer = _make_sc_layer(False)


def _pack_bf16_pairs(h):
    lo = h[:64, :].astype(jnp.bfloat16)
    hi = h[64:, :].astype(jnp.bfloat16)
    lob = lax.bitcast_convert_type(lo, jnp.uint16).astype(jnp.uint32)
    hib = lax.bitcast_convert_type(hi, jnp.uint16).astype(jnp.uint32)
    return lax.bitcast_convert_type(lob | (hib << 16), jnp.int32)


def _tcA_body(x_in, dego, degi, h_out, ns_out, nd_out):
    deg_o = jnp.sum(dego[...], axis=0, keepdims=True)
    deg_i = jnp.sum(degi[...], axis=0, keepdims=True)
    ns = jnp.where(deg_o > 0, lax.rsqrt(jnp.maximum(deg_o, 1.0)), 0.0)
    nd = jnp.where(deg_i > 0, lax.rsqrt(jnp.maximum(deg_i, 1.0)), 0.0)
    ns_out[...] = ns
    nd_out[...] = nd
    inT = jnp.transpose(x_in[...], (1, 0))
    h_out[...] = _pack_bf16_pairs(inT * ns)


def _tc_norms(inT, dego_p, degi_p):
    return pl.pallas_call(
        _tcA_body,
        out_shape=[
            jax.ShapeDtypeStruct((FF // 2, NN), jnp.int32),
            jax.ShapeDtypeStruct((1, NN), jnp.float32),
            jax.ShapeDtypeStruct((1, NN), jnp.float32),
        ],
    )(inT, dego_p, degi_p)


def _tcB_body(agg, nd, ns, W, b, h_out):
    x = agg[...] * nd[...]
    y = lax.dot_general(W[...], x, (((0,), (0,)), ((), ())),
                        preferred_element_type=jnp.float32)
    h_out[...] = _pack_bf16_pairs(jnp.maximum(y + b[...], 0.0) * ns[...])


def _tc_layer(aggT, nd, ns, W, b2d):
    return pl.pallas_call(
        _tcB_body,
        out_shape=jax.ShapeDtypeStruct((FF // 2, NN), jnp.int32),
    )(aggT, nd, ns, W, b2d)


def _tcC_body(agg, nd, wsp, ns, W2, b2, W3, b3, out):
    x = agg[...] * nd[...]
    y = lax.dot_general(W2[...], x, (((0,), (0,)), ((), ())),
                        preferred_element_type=jnp.float32)
    h2 = jnp.maximum(y + b2[...], 0.0)
    wv = ns[...] * jnp.sum(wsp[...], axis=0, keepdims=True)
    g = lax.dot_general(h2, wv, (((1,), (1,)), ((), ())),
                        preferred_element_type=jnp.float32) * (1.0 / NN)
    p = lax.dot_general(W3[...], g, (((0,), (0,)), ((), ())),
                        preferred_element_type=jnp.float32) + b3[...]
    m = jnp.max(p, axis=0, keepdims=True)
    e = jnp.exp(p - m)
    out[...] = e / jnp.sum(e, axis=0, keepdims=True)


def _tc_final(aggT, nd, wsum_p, ns, W2, b2d, W3, b3d):
    return pl.pallas_call(
        _tcC_body,
        out_shape=jax.ShapeDtypeStruct((CC, 1), jnp.float32),
    )(aggT, nd, wsum_p, ns, W2, b2d, W3, b3d)


def kernel(in_feat, edge_index, W1, b1, W2, b2, W3, b3):
    ep, dego_p, degi_p = _sc_pack_deg(edge_index)
    h0sT, ns, nd = _tc_norms(in_feat, dego_p, degi_p)
    aggT1, wsum_p = _sc_layer_wsum(h0sT, ep, nd.reshape(NN))
    h1sT = _tc_layer(aggT1, nd, ns, W1, b1.reshape(FF, 1))
    aggT2 = _sc_layer(h1sT, ep)
    out = _tc_final(aggT2, nd, wsum_p, ns, W2, b2.reshape(FF, 1),
                    W3, b3.reshape(CC, 1))
    return out.reshape(CC)
